# Initial kernel scaffold; baseline (speedup 1.0000x reference)
import jax
import jax.numpy as jnp
from jax.experimental import pallas as pl


def kernel(x0, x1, x2, parent1, parent0, W, U, b):
    raise NotImplementedError("write your pallas kernel here")


# speedup vs baseline: 1.0450x; 1.0450x over previous
import functools

import jax
import jax.numpy as jnp
from jax import lax
from jax.experimental import pallas as pl
from jax.experimental.pallas import tpu as pltpu
from jax.experimental.pallas import tpu_sc as plsc

INPUT = 128
H = 128
N2, N1, N0 = 262144, 16384, 1024
ROWS_TOTAL = N0 + N1 + N2
OFF1 = N0 + N1
OFF0 = N0
BLK = 512


def _leaf_body(x_ref, w3_ref, b3_ref, out_ref, c_ref):
    g = jnp.dot(x_ref[...], w3_ref[...], preferred_element_type=jnp.float32)
    g = g + b3_ref[...]
    i = jax.nn.sigmoid(g[:, 0:H])
    o = jax.nn.sigmoid(g[:, H:2 * H])
    u = jnp.tanh(g[:, 2 * H:3 * H])
    c = i * u
    out_ref[...] = o * jnp.tanh(c)
    c_ref[...] = c


def _leaf_call(x2, w3, b3):
    grid = N2 // BLK
    return pl.pallas_call(
        _leaf_body,
        grid=(grid,),
        in_specs=[
            pl.BlockSpec((BLK, INPUT), lambda i: (i, 0)),
            pl.BlockSpec((INPUT, 3 * H), lambda i: (0, 0)),
            pl.BlockSpec((1, 3 * H), lambda i: (0, 0)),
        ],
        out_specs=[
            pl.BlockSpec((BLK, H), lambda i: (OFF1 // BLK + i, 0)),
            pl.BlockSpec((BLK, H), lambda i: (i, 0)),
        ],
        out_shape=[
            jax.ShapeDtypeStruct((ROWS_TOTAL, H), jnp.float32),
            jax.ShapeDtypeStruct((N2, H), jnp.float32),
        ],
    )(x2, w3, b3)


def _cell_body(full_ref, x_ref, hs_ref, cs_ref, w_ref, u_ref, b_ref,
               out_ref, c_ref):
    del full_ref
    g = jnp.dot(x_ref[...], w_ref[...], preferred_element_type=jnp.float32)
    g = g + jnp.dot(hs_ref[...], u_ref[...], preferred_element_type=jnp.float32)
    g = g + b_ref[...]
    i = jax.nn.sigmoid(g[:, 0:H])
    f = jax.nn.sigmoid(g[:, H:2 * H])
    o = jax.nn.sigmoid(g[:, 2 * H:3 * H])
    u = jnp.tanh(g[:, 3 * H:4 * H])
    c = f * cs_ref[...] + i * u
    out_ref[...] = o * jnp.tanh(c)
    c_ref[...] = c


def _cell_call(full, x, hs, cs, w, u, b4, n, row_off, want_c):
    grid = n // BLK
    out_specs = [pl.BlockSpec((BLK, H), lambda i: (row_off // BLK + i, 0))]
    out_shape = [jax.ShapeDtypeStruct((ROWS_TOTAL, H), jnp.float32)]
    if want_c:
        out_specs.append(pl.BlockSpec((BLK, H), lambda i: (i, 0)))
        out_shape.append(jax.ShapeDtypeStruct((n, H), jnp.float32))
        body = _cell_body
    else:
        def body(full_ref, x_ref, hs_ref, cs_ref, w_ref, u_ref, b_ref, out_ref):
            del full_ref
            g = jnp.dot(x_ref[...], w_ref[...], preferred_element_type=jnp.float32)
            g = g + jnp.dot(hs_ref[...], u_ref[...],
                            preferred_element_type=jnp.float32)
            g = g + b_ref[...]
            i = jax.nn.sigmoid(g[:, 0:H])
            f = jax.nn.sigmoid(g[:, H:2 * H])
            o = jax.nn.sigmoid(g[:, 2 * H:3 * H])
            uu = jnp.tanh(g[:, 3 * H:4 * H])
            c = f * cs_ref[...] + i * uu
            out_ref[...] = o * jnp.tanh(c)
    out = pl.pallas_call(
        body,
        grid=(grid,),
        in_specs=[
            pl.BlockSpec(memory_space=pltpu.MemorySpace.HBM),
            pl.BlockSpec((BLK, INPUT), lambda i: (i, 0)),
            pl.BlockSpec((BLK, H), lambda i: (i, 0)),
            pl.BlockSpec((BLK, H), lambda i: (i, 0)),
            pl.BlockSpec((INPUT, 4 * H), lambda i: (0, 0)),
            pl.BlockSpec((H, 4 * H), lambda i: (0, 0)),
            pl.BlockSpec((1, 4 * H), lambda i: (0, 0)),
        ],
        out_specs=out_specs,
        out_shape=out_shape,
        input_output_aliases={0: 0},
    )(full, x, hs, cs, w, u, b4)
    return out if want_c else out[0]


_SC_CH = 1024
_SC_HALF = 512


def _segsum2_sc(src_h, off_h, src_c, off_c, parents3d, m, n):
    ch = _SC_CH
    half = _SC_HALF
    nsub = 16
    nph = 4 if m > 2048 else 1
    per_sub = n // nsub
    n_chunks = per_sub // ch
    m_q = m // (2 * nph)
    m16 = m_q // nsub
    acc_rows = m_q + 8
    mesh = plsc.VectorSubcoreMesh(core_axis_name="c", subcore_axis_name="s")

    @functools.partial(
        pl.kernel,
        out_type=[jax.ShapeDtypeStruct((m, H), jnp.float32),
                  jax.ShapeDtypeStruct((m, H), jnp.float32)],
        mesh=mesh,
        scratch_types=[
            pltpu.VMEM((8, 128), jnp.int32),
            pltpu.VMEM((half, H), jnp.float32),
            pltpu.VMEM((min(half, max(8, m_q // nsub)), H), jnp.float32),
            pltpu.VMEM_SHARED((acc_rows, H), jnp.float32),
        ],
    )
    def k(srch_hbm, srcc_hbm, par_hbm, outh_hbm, outc_hbm,
          idx_v, rows_v, zbuf, acc):
        core = lax.axis_index("c")
        sub = lax.axis_index("s")

        zrows = min(half, max(8, m16))

        def zb(t, carry):
            zbuf[t // 8, pl.ds((t % 8) * 16, 16)] = jnp.zeros((16,), jnp.float32)
            return carry
        lax.fori_loop(0, zrows * 8, zb, 0)

        def run(src_hbm, out_hbm, row_off):
            for ph in range(nph):
                lo = (core * nph + ph) * m_q
                for kk in range(max(1, m16 // zrows)):
                    pltpu.sync_copy(zbuf.at[pl.ds(0, min(zrows, m16))],
                                    acc.at[pl.ds(sub * m16 + kk * zrows,
                                                 min(zrows, m16))])

                @pl.when(sub == 0)
                def _zero_dump():
                    pltpu.sync_copy(zbuf.at[pl.ds(0, 8)], acc.at[pl.ds(m_q, 8)])
                plsc.subcore_barrier()

                def chunk(ci, carry):
                    blk = sub * n_chunks + ci
                    pltpu.sync_copy(par_hbm.at[blk], idx_v)
                    cmin = idx_v[0, pl.ds(0, 16)][0]
                    cmax = idx_v[7, pl.ds(112, 16)][15]

                    def do_chunk():
                        def adj(t, c2):
                            r = t // 8
                            s16 = (t % 8) * 16
                            v = idx_v[r, pl.ds(s16, 16)] - lo
                            bad = (v < 0) | (v >= m_q)
                            idx_v[r, pl.ds(s16, 16)] = jnp.where(bad, m_q, v)
                            return c2
                        lax.fori_loop(0, 64, adj, 0)
                        for hh in range(ch // half):
                            start = pl.multiple_of(
                                row_off + blk * ch + hh * half, half)
                            pltpu.sync_copy(src_hbm.at[pl.ds(start, half)],
                                            rows_v)
                            for j in range(half // 128):
                                pltpu.sync_copy(
                                    rows_v.at[pl.ds(j * 128, 128)],
                                    acc.at[idx_v.at[hh * (half // 128) + j]],
                                    add=True)
                    lax.cond((cmin < lo + m_q) & (cmax >= lo),
                             do_chunk, lambda: None)
                    return carry
                lax.fori_loop(0, n_chunks, chunk, 0)
                plsc.subcore_barrier()
                pltpu.sync_copy(
                    acc.at[pl.ds(pl.multiple_of(sub * m16, 8), m16)],
                    out_hbm.at[pl.ds(pl.multiple_of(lo + sub * m16, 8), m16)])
                plsc.subcore_barrier()

        run(srch_hbm, outh_hbm, off_h)
        run(srcc_hbm, outc_hbm, off_c)

    return k(src_h, src_c, parents3d)


def kernel(x0, x1, x2, parent1, parent0, W, U, b):
    p1 = parent1.astype(jnp.int32).reshape(N2 // _SC_CH, 8, 128)
    p0 = parent0.astype(jnp.int32).reshape(N1 // _SC_CH, 8, 128)
    w3 = jnp.concatenate([W[:, 0:H], W[:, 2 * H:4 * H]], axis=1)
    b3 = jnp.concatenate([b[0:H], b[2 * H:4 * H]]).reshape(1, 3 * H)
    b4 = b.reshape(1, 4 * H)

    full, c2 = _leaf_call(x2, w3, b3)
    h_sum1, c_sum1 = _segsum2_sc(full, OFF1, c2, 0, p1, N1, N2)
    full, c1 = _cell_call(full, x1, h_sum1, c_sum1, W, U, b4, N1, OFF0, True)
    h_sum0, c_sum0 = _segsum2_sc(full, OFF0, c1, 0, p0, N0, N1)
    out = _cell_call(full, x0, h_sum0, c_sum0, W, U, b4, N0, 0, False)
    return out


# speedup vs baseline: 2.7339x; 2.6161x over previous
import functools

import jax
import jax.numpy as jnp
from jax import lax
from jax.experimental import pallas as pl
from jax.experimental.pallas import tpu as pltpu
from jax.experimental.pallas import tpu_sc as plsc

INPUT = 128
H = 128
N2, N1, N0 = 262144, 16384, 1024
ROWS_TOTAL = N0 + N1 + N2
OFF1 = N0 + N1
OFF0 = N0
BLK = 512


def _leaf_body(x_ref, w3_ref, b3_ref, out_ref, c_ref):
    g = jnp.dot(x_ref[...], w3_ref[...], preferred_element_type=jnp.float32)
    g = g + b3_ref[...]
    i = jax.nn.sigmoid(g[:, 0:H])
    o = jax.nn.sigmoid(g[:, H:2 * H])
    u = jnp.tanh(g[:, 2 * H:3 * H])
    c = i * u
    out_ref[...] = o * jnp.tanh(c)
    c_ref[...] = c


def _leaf_call(x2, w3, b3):
    grid = N2 // BLK
    return pl.pallas_call(
        _leaf_body,
        grid=(grid,),
        in_specs=[
            pl.BlockSpec((BLK, INPUT), lambda i: (i, 0)),
            pl.BlockSpec((INPUT, 3 * H), lambda i: (0, 0)),
            pl.BlockSpec((1, 3 * H), lambda i: (0, 0)),
        ],
        out_specs=[
            pl.BlockSpec((BLK, H), lambda i: (OFF1 // BLK + i, 0)),
            pl.BlockSpec((BLK, H), lambda i: (i, 0)),
        ],
        out_shape=[
            jax.ShapeDtypeStruct((ROWS_TOTAL, H), jnp.float32),
            jax.ShapeDtypeStruct((N2, H), jnp.float32),
        ],
    )(x2, w3, b3)


def _cell_body(full_ref, x_ref, hs_ref, cs_ref, w_ref, u_ref, b_ref,
               out_ref, c_ref):
    del full_ref
    g = jnp.dot(x_ref[...], w_ref[...], preferred_element_type=jnp.float32)
    g = g + jnp.dot(hs_ref[...], u_ref[...], preferred_element_type=jnp.float32)
    g = g + b_ref[...]
    i = jax.nn.sigmoid(g[:, 0:H])
    f = jax.nn.sigmoid(g[:, H:2 * H])
    o = jax.nn.sigmoid(g[:, 2 * H:3 * H])
    u = jnp.tanh(g[:, 3 * H:4 * H])
    c = f * cs_ref[...] + i * u
    out_ref[...] = o * jnp.tanh(c)
    c_ref[...] = c


def _cell_call(full, x, hs, cs, w, u, b4, n, row_off, want_c):
    grid = n // BLK
    out_specs = [pl.BlockSpec((BLK, H), lambda i: (row_off // BLK + i, 0))]
    out_shape = [jax.ShapeDtypeStruct((ROWS_TOTAL, H), jnp.float32)]
    if want_c:
        out_specs.append(pl.BlockSpec((BLK, H), lambda i: (i, 0)))
        out_shape.append(jax.ShapeDtypeStruct((n, H), jnp.float32))
        body = _cell_body
    else:
        def body(full_ref, x_ref, hs_ref, cs_ref, w_ref, u_ref, b_ref, out_ref):
            del full_ref
            g = jnp.dot(x_ref[...], w_ref[...], preferred_element_type=jnp.float32)
            g = g + jnp.dot(hs_ref[...], u_ref[...],
                            preferred_element_type=jnp.float32)
            g = g + b_ref[...]
            i = jax.nn.sigmoid(g[:, 0:H])
            f = jax.nn.sigmoid(g[:, H:2 * H])
            o = jax.nn.sigmoid(g[:, 2 * H:3 * H])
            uu = jnp.tanh(g[:, 3 * H:4 * H])
            c = f * cs_ref[...] + i * uu
            out_ref[...] = o * jnp.tanh(c)
    out = pl.pallas_call(
        body,
        grid=(grid,),
        in_specs=[
            pl.BlockSpec(memory_space=pltpu.MemorySpace.HBM),
            pl.BlockSpec((BLK, INPUT), lambda i: (i, 0)),
            pl.BlockSpec((BLK, H), lambda i: (i, 0)),
            pl.BlockSpec((BLK, H), lambda i: (i, 0)),
            pl.BlockSpec((INPUT, 4 * H), lambda i: (0, 0)),
            pl.BlockSpec((H, 4 * H), lambda i: (0, 0)),
            pl.BlockSpec((1, 4 * H), lambda i: (0, 0)),
        ],
        out_specs=out_specs,
        out_shape=out_shape,
        input_output_aliases={0: 0},
    )(full, x, hs, cs, w, u, b4)
    return out if want_c else out[0]


_SC_CH = 1024
_SC_HALF = 512


def _segsum2_sc(src_h, off_h, src_c, off_c, parents3d, m, n):
    ch = _SC_CH
    half = _SC_HALF
    nsub = 16
    nph = 4 if m > 2048 else 1
    per_sub = n // nsub
    n_chunks = per_sub // ch
    m_q = m // (2 * nph)
    m16 = m_q // nsub
    acc_rows = m_q + 8
    mesh = plsc.VectorSubcoreMesh(core_axis_name="c", subcore_axis_name="s")

    ub = 128
    units = ch // ub
    zrows = min(64, max(8, m16))

    @functools.partial(
        pl.kernel,
        out_type=[jax.ShapeDtypeStruct((m, H), jnp.float32),
                  jax.ShapeDtypeStruct((m, H), jnp.float32)],
        mesh=mesh,
        scratch_types=[
            pltpu.VMEM((2, 8, 128), jnp.int32),
            pltpu.VMEM((3, ub, H), jnp.float32),
            pltpu.VMEM((zrows, H), jnp.float32),
            pltpu.VMEM_SHARED((acc_rows, H), jnp.float32),
            pltpu.SemaphoreType.DMA,
            pltpu.SemaphoreType.DMA,
            pltpu.SemaphoreType.DMA,
            pltpu.SemaphoreType.DMA,
            pltpu.SemaphoreType.DMA,
            pltpu.SemaphoreType.DMA,
            pltpu.SemaphoreType.DMA,
            pltpu.SemaphoreType.DMA,
        ],
    )
    def k(srch_hbm, srcc_hbm, par_hbm, outh_hbm, outc_hbm,
          idx2, rows3, zbuf, acc, si0, si1, sr0, sr1, sr2, ss0, ss1, ss2):
        core = lax.axis_index("c")
        sub = lax.axis_index("s")
        sem_i = [si0, si1]
        sem_r = [sr0, sr1, sr2]
        sem_s = [ss0, ss1, ss2]

        def zb(t, carry):
            zbuf[t // 8, pl.ds((t % 8) * 16, 16)] = jnp.zeros((16,), jnp.float32)
            return carry
        lax.fori_loop(0, zrows * 8, zb, 0)

        def process_chunk(src_hbm, row_off, blk, q, lo):
            base = pl.multiple_of(row_off + blk * ch, ub)
            loads = {}
            for hh in range(min(3, units)):
                loads[hh] = pltpu.async_copy(
                    src_hbm.at[pl.ds(base + hh * ub, ub)],
                    rows3.at[hh], sem_r[hh])

            def adj(t, c2):
                r = t // 8
                s16 = (t % 8) * 16
                v = idx2[q, r, pl.ds(s16, 16)] - lo
                bad = (v < 0) | (v >= m_q)
                idx2[q, r, pl.ds(s16, 16)] = jnp.where(bad, m_q, v)
                return c2
            lax.fori_loop(0, 64, adj, 0)

            tail = []
            for hh in range(units):
                b = hh % 3
                loads[hh].wait()
                pair = []
                for j in range(ub // 128):
                    pair.append(pltpu.async_copy(
                        rows3.at[b, pl.ds(j * 128, 128)],
                        acc.at[idx2.at[q, hh * (ub // 128) + j]],
                        sem_s[b], add=True))
                if hh + 3 < units:
                    for s in pair:
                        s.wait()
                    loads[hh + 3] = pltpu.async_copy(
                        src_hbm.at[pl.ds(base + (hh + 3) * ub, ub)],
                        rows3.at[b], sem_r[b])
                else:
                    tail.extend(pair)
            for s in tail:
                s.wait()

        def run(src_hbm, out_hbm, row_off):
            for ph in range(nph):
                lo = (core * nph + ph) * m_q
                for kk in range(max(1, m16 // zrows)):
                    pltpu.sync_copy(zbuf.at[pl.ds(0, min(zrows, m16))],
                                    acc.at[pl.ds(sub * m16 + kk * zrows,
                                                 min(zrows, m16))])

                @pl.when(sub == 0)
                def _zero_dump():
                    pltpu.sync_copy(zbuf.at[pl.ds(0, 8)], acc.at[pl.ds(m_q, 8)])
                plsc.subcore_barrier()

                if n_chunks == 1:
                    blk = sub
                    pltpu.sync_copy(par_hbm.at[blk], idx2.at[0])
                    cmin = idx2[0, 0, pl.ds(0, 16)][0]
                    cmax = idx2[0, 7, pl.ds(112, 16)][15]
                    lax.cond((cmin < lo + m_q) & (cmax >= lo),
                             lambda: process_chunk(src_hbm, row_off, blk, 0, lo),
                             lambda: None)
                else:
                    pltpu.async_copy(par_hbm.at[sub], idx2.at[0], sem_i[0])

                    def pair_body(t, carry):
                        for q in range(2):
                            ci = 2 * t + q
                            blk = ci * nsub + sub
                            pltpu.make_async_copy(
                                par_hbm.at[blk], idx2.at[q], sem_i[q]).wait()

                            @pl.when(ci + 1 < n_chunks)
                            def _prefetch():
                                pltpu.async_copy(
                                    par_hbm.at[blk + nsub], idx2.at[1 - q],
                                    sem_i[1 - q])
                            cmin = idx2[q, 0, pl.ds(0, 16)][0]
                            cmax = idx2[q, 7, pl.ds(112, 16)][15]
                            qq = q
                            bb = blk
                            lax.cond(
                                (cmin < lo + m_q) & (cmax >= lo),
                                functools.partial(process_chunk, src_hbm,
                                                  row_off, bb, qq, lo),
                                lambda: None)
                        return carry
                    lax.fori_loop(0, n_chunks // 2, pair_body, 0)
                plsc.subcore_barrier()
                pltpu.sync_copy(
                    acc.at[pl.ds(pl.multiple_of(sub * m16, 8), m16)],
                    out_hbm.at[pl.ds(pl.multiple_of(lo + sub * m16, 8), m16)])
                plsc.subcore_barrier()

        run(srch_hbm, outh_hbm, off_h)
        run(srcc_hbm, outc_hbm, off_c)

    return k(src_h, src_c, parents3d)


def kernel(x0, x1, x2, parent1, parent0, W, U, b):
    p1 = parent1.astype(jnp.int32).reshape(N2 // _SC_CH, 8, 128)
    p0 = parent0.astype(jnp.int32).reshape(N1 // _SC_CH, 8, 128)
    w3 = jnp.concatenate([W[:, 0:H], W[:, 2 * H:4 * H]], axis=1)
    b3 = jnp.concatenate([b[0:H], b[2 * H:4 * H]]).reshape(1, 3 * H)
    b4 = b.reshape(1, 4 * H)

    full, c2 = _leaf_call(x2, w3, b3)
    h_sum1, c_sum1 = _segsum2_sc(full, OFF1, c2, 0, p1, N1, N2)
    full, c1 = _cell_call(full, x1, h_sum1, c_sum1, W, U, b4, N1, OFF0, True)
    h_sum0, c_sum0 = _segsum2_sc(full, OFF0, c1, 0, p0, N0, N1)
    out = _cell_call(full, x0, h_sum0, c_sum0, W, U, b4, N0, 0, False)
    return out


# speedup vs baseline: 3.5641x; 1.3037x over previous
import functools

import jax
import jax.numpy as jnp
from jax import lax
from jax.experimental import pallas as pl
from jax.experimental.pallas import tpu as pltpu
from jax.experimental.pallas import tpu_sc as plsc

INPUT = 128
H = 128
N2, N1, N0 = 262144, 16384, 1024
ROWS_TOTAL = N0 + N1 + N2
OFF1 = N0 + N1
OFF0 = N0
BLK = 512


def _leaf_body(x_ref, w3_ref, b3_ref, out_ref, c_ref):
    g = jnp.dot(x_ref[...], w3_ref[...], preferred_element_type=jnp.float32)
    g = g + b3_ref[...]
    i = jax.nn.sigmoid(g[:, 0:H])
    o = jax.nn.sigmoid(g[:, H:2 * H])
    u = jnp.tanh(g[:, 2 * H:3 * H])
    c = i * u
    out_ref[...] = o * jnp.tanh(c)
    c_ref[...] = c


LEAF_BLK = 1024


def _leaf_call(x2, w3, b3):
    grid = N2 // LEAF_BLK
    return pl.pallas_call(
        _leaf_body,
        grid=(grid,),
        in_specs=[
            pl.BlockSpec((LEAF_BLK, INPUT), lambda i: (i, 0)),
            pl.BlockSpec((INPUT, 3 * H), lambda i: (0, 0)),
            pl.BlockSpec((1, 3 * H), lambda i: (0, 0)),
        ],
        out_specs=[
            pl.BlockSpec((LEAF_BLK, H), lambda i: (OFF1 // LEAF_BLK + i, 0)),
            pl.BlockSpec((LEAF_BLK, H), lambda i: (i, 0)),
        ],
        out_shape=[
            jax.ShapeDtypeStruct((ROWS_TOTAL, H), jnp.float32),
            jax.ShapeDtypeStruct((N2, H), jnp.float32),
        ],
    )(x2, w3, b3)


def _cell_body(full_ref, x_ref, hs_ref, cs_ref, w_ref, u_ref, b_ref,
               out_ref, c_ref):
    del full_ref
    g = jnp.dot(x_ref[...], w_ref[...], preferred_element_type=jnp.float32)
    g = g + jnp.dot(hs_ref[...], u_ref[...], preferred_element_type=jnp.float32)
    g = g + b_ref[...]
    i = jax.nn.sigmoid(g[:, 0:H])
    f = jax.nn.sigmoid(g[:, H:2 * H])
    o = jax.nn.sigmoid(g[:, 2 * H:3 * H])
    u = jnp.tanh(g[:, 3 * H:4 * H])
    c = f * cs_ref[...] + i * u
    out_ref[...] = o * jnp.tanh(c)
    c_ref[...] = c


def _cell_call(full, x, hs, cs, w, u, b4, n, row_off, want_c):
    grid = n // BLK
    out_specs = [pl.BlockSpec((BLK, H), lambda i: (row_off // BLK + i, 0))]
    out_shape = [jax.ShapeDtypeStruct((ROWS_TOTAL, H), jnp.float32)]
    if want_c:
        out_specs.append(pl.BlockSpec((BLK, H), lambda i: (i, 0)))
        out_shape.append(jax.ShapeDtypeStruct((n, H), jnp.float32))
        body = _cell_body
    else:
        def body(full_ref, x_ref, hs_ref, cs_ref, w_ref, u_ref, b_ref, out_ref):
            del full_ref
            g = jnp.dot(x_ref[...], w_ref[...], preferred_element_type=jnp.float32)
            g = g + jnp.dot(hs_ref[...], u_ref[...],
                            preferred_element_type=jnp.float32)
            g = g + b_ref[...]
            i = jax.nn.sigmoid(g[:, 0:H])
            f = jax.nn.sigmoid(g[:, H:2 * H])
            o = jax.nn.sigmoid(g[:, 2 * H:3 * H])
            uu = jnp.tanh(g[:, 3 * H:4 * H])
            c = f * cs_ref[...] + i * uu
            out_ref[...] = o * jnp.tanh(c)
    out = pl.pallas_call(
        body,
        grid=(grid,),
        in_specs=[
            pl.BlockSpec(memory_space=pltpu.MemorySpace.HBM),
            pl.BlockSpec((BLK, INPUT), lambda i: (i, 0)),
            pl.BlockSpec((BLK, H), lambda i: (i, 0)),
            pl.BlockSpec((BLK, H), lambda i: (i, 0)),
            pl.BlockSpec((INPUT, 4 * H), lambda i: (0, 0)),
            pl.BlockSpec((H, 4 * H), lambda i: (0, 0)),
            pl.BlockSpec((1, 4 * H), lambda i: (0, 0)),
        ],
        out_specs=out_specs,
        out_shape=out_shape,
        input_output_aliases={0: 0},
    )(full, x, hs, cs, w, u, b4)
    return out if want_c else out[0]


_SC_CH = 1024
_SC_HALF = 512


def _segsum2_sc(src_h, off_h, src_c, off_c, parents3d, m, n):
    ch = _SC_CH
    half = _SC_HALF
    nsub = 16
    nph = 4 if m > 2048 else 1
    per_sub = n // nsub
    n_chunks = per_sub // ch
    m_q = m // (2 * nph)
    m16 = m_q // nsub
    acc_rows = m_q + 8
    mesh = plsc.VectorSubcoreMesh(core_axis_name="c", subcore_axis_name="s")

    ub = 128
    units = ch // ub
    zrows = min(64, max(8, m16))

    @functools.partial(
        pl.kernel,
        out_type=[jax.ShapeDtypeStruct((m, H), jnp.float32),
                  jax.ShapeDtypeStruct((m, H), jnp.float32)],
        mesh=mesh,
        scratch_types=[
            pltpu.VMEM((8, 128), jnp.int32),
            pltpu.VMEM((3, ub, H), jnp.float32),
            pltpu.VMEM((zrows, H), jnp.float32),
            pltpu.VMEM_SHARED((acc_rows, H), jnp.float32),
            pltpu.SemaphoreType.DMA,
            pltpu.SemaphoreType.DMA,
            pltpu.SemaphoreType.DMA,
            pltpu.SemaphoreType.DMA,
        ],
    )
    def k(srch_hbm, srcc_hbm, par_hbm, outh_hbm, outc_hbm,
          idx_v, rows3, zbuf, acc, sr0, sr1, sr2, ss):
        core = lax.axis_index("c")
        sub = lax.axis_index("s")
        sem_r = [sr0, sr1, sr2]

        def zb(t, carry):
            zbuf[t // 8, pl.ds((t % 8) * 16, 16)] = jnp.zeros((16,), jnp.float32)
            return carry
        lax.fori_loop(0, zrows * 8, zb, 0)

        def process_chunk(src_hbm, row_off, blk, lo):
            base = pl.multiple_of(row_off + blk * ch, ub)
            loads = {}
            for hh in range(min(3, units)):
                loads[hh] = pltpu.async_copy(
                    src_hbm.at[pl.ds(base + hh * ub, ub)],
                    rows3.at[hh], sem_r[hh])

            def adj(t, c2):
                r = t // 8
                s16 = (t % 8) * 16
                v = idx_v[r, pl.ds(s16, 16)] - lo
                bad = (v < 0) | (v >= m_q)
                idx_v[r, pl.ds(s16, 16)] = jnp.where(bad, m_q, v)
                return c2
            lax.fori_loop(0, 64, adj, 0)

            tail = []
            for hh in range(units):
                b = hh % 3
                loads[hh].wait()
                pair = []
                for j in range(ub // 128):
                    pair.append(pltpu.async_copy(
                        rows3.at[b, pl.ds(j * 128, 128)],
                        acc.at[idx_v.at[hh * (ub // 128) + j]],
                        ss, add=True))
                if hh + 3 < units:
                    for s in pair:
                        s.wait()
                    loads[hh + 3] = pltpu.async_copy(
                        src_hbm.at[pl.ds(base + (hh + 3) * ub, ub)],
                        rows3.at[b], sem_r[b])
                else:
                    tail.extend(pair)
            for s in tail:
                s.wait()

        def run(src_hbm, out_hbm, row_off):
            pos = jnp.int32(0)
            for ph in range(nph):
                lo = (core * nph + ph) * m_q
                hi = lo + m_q
                for kk in range(max(1, m16 // zrows)):
                    pltpu.sync_copy(zbuf.at[pl.ds(0, min(zrows, m16))],
                                    acc.at[pl.ds(sub * m16 + kk * zrows,
                                                 min(zrows, m16))])

                @pl.when(sub == 0)
                def _zero_dump():
                    pltpu.sync_copy(zbuf.at[pl.ds(0, 8)], acc.at[pl.ds(m_q, 8)])
                plsc.subcore_barrier()

                def chunk_body(ci, st):
                    p, stopped = st
                    take = (ci == p) & jnp.logical_not(stopped)

                    def active():
                        blk = ci * nsub + sub
                        pltpu.sync_copy(par_hbm.at[blk], idx_v)
                        cmin = idx_v[0, pl.ds(0, 16)][0]
                        cmax = idx_v[7, pl.ds(112, 16)][15]
                        lax.cond((cmin < hi) & (cmax >= lo),
                                 functools.partial(process_chunk, src_hbm,
                                                   row_off, blk, lo),
                                 lambda: None)
                        consumed = cmax < hi
                        return (jnp.where(consumed, p + 1, p),
                                jnp.logical_not(consumed))

                    def idle():
                        return p, stopped
                    return lax.cond(take, active, idle)
                pos, _ = lax.fori_loop(0, n_chunks, chunk_body,
                                       (pos, jnp.bool_(False)))
                plsc.subcore_barrier()
                pltpu.sync_copy(
                    acc.at[pl.ds(pl.multiple_of(sub * m16, 8), m16)],
                    out_hbm.at[pl.ds(pl.multiple_of(lo + sub * m16, 8), m16)])
                plsc.subcore_barrier()

        run(srch_hbm, outh_hbm, off_h)
        run(srcc_hbm, outc_hbm, off_c)

    return k(src_h, src_c, parents3d)


def kernel(x0, x1, x2, parent1, parent0, W, U, b):
    p1 = parent1.astype(jnp.int32).reshape(N2 // _SC_CH, 8, 128)
    p0 = parent0.astype(jnp.int32).reshape(N1 // _SC_CH, 8, 128)
    w3 = jnp.concatenate([W[:, 0:H], W[:, 2 * H:4 * H]], axis=1)
    b3 = jnp.concatenate([b[0:H], b[2 * H:4 * H]]).reshape(1, 3 * H)
    b4 = b.reshape(1, 4 * H)

    full, c2 = _leaf_call(x2, w3, b3)
    h_sum1, c_sum1 = _segsum2_sc(full, OFF1, c2, 0, p1, N1, N2)
    full, c1 = _cell_call(full, x1, h_sum1, c_sum1, W, U, b4, N1, OFF0, True)
    h_sum0, c_sum0 = _segsum2_sc(full, OFF0, c1, 0, p0, N0, N1)
    out = _cell_call(full, x0, h_sum0, c_sum0, W, U, b4, N0, 0, False)
    return out
